# baseline (speedup 1.0000x reference)
import functools

import jax
import jax.numpy as jnp
import numpy as np
from jax.experimental import pallas as pl
from jax.experimental.pallas import tpu as pltpu
from jax.experimental.pallas import tpu_sc as plsc

_D = 64
_SEQ = 200
_SCALE = 8.0

_GATHER_WINDOW = 128
_TC_ROWS = 128


def _pe_table() -> np.ndarray:
    half = _D / 2
    positions = np.arange(_SEQ)[:, np.newaxis]
    depths = np.arange(half)[np.newaxis, :] / half
    angle_rads = positions * (1.0 / 10000**depths)
    return np.concatenate(
        [np.sin(angle_rads), np.cos(angle_rads)], axis=-1
    ).astype(np.float32)


def _sc_gather(table, idx):
    n = idx.shape[1]
    mesh = plsc.VectorSubcoreMesh(core_axis_name="core", subcore_axis_name="subcore")

    @functools.partial(
        pl.kernel,
        out_type=jax.ShapeDtypeStruct((n, _D), table.dtype),
        mesh=mesh,
        compiler_params=pltpu.CompilerParams(use_tc_tiling_on_sc=False),
    )
    def k(table_hbm, i_hbm, o_hbm):
        def body(i_vmem, o_vmem):
            pltpu.sync_copy(table_hbm.at[i_vmem.at[0]], o_vmem)

        pltpu.emit_pipeline(
            body,
            grid=(n // _GATHER_WINDOW,),
            in_specs=[pl.BlockSpec((1, _GATHER_WINDOW), index_map=lambda i: (0, i))],
            out_specs=[pl.BlockSpec((_GATHER_WINDOW, _D), index_map=lambda i: (i, 0))],
            core_axis_name=("core", "subcore"),
            dimension_semantics=(pltpu.PARALLEL,),
        )(i_hbm, o_hbm)

    return k(table, idx)


def _fixup_body(g_ref, pe_ref, o_ref):
    o_ref[...] = g_ref[...] * _SCALE + pe_ref[...]


def _tc_fixup(g, pe_flat):
    b = g.shape[0]
    w = _SEQ * _D
    return pl.pallas_call(
        _fixup_body,
        grid=(b // _TC_ROWS,),
        in_specs=[
            pl.BlockSpec((_TC_ROWS, w), lambda i: (i, 0)),
            pl.BlockSpec((1, w), lambda i: (0, 0)),
        ],
        out_specs=pl.BlockSpec((_TC_ROWS, w), lambda i: (i, 0)),
        out_shape=jax.ShapeDtypeStruct((b, w), jnp.float32),
    )(g, pe_flat)


def kernel(x, table):
    b, l = x.shape
    idx = x.reshape(1, b * l).astype(jnp.int32)
    g = _sc_gather(table, idx)
    pe_flat = jnp.asarray(_pe_table().reshape(1, l * _D))
    out = _tc_fixup(g.reshape(b, l * _D), pe_flat)
    return out.reshape(b, l, _D)
